# Initial kernel scaffold; baseline (speedup 1.0000x reference)
import jax
import jax.numpy as jnp
from jax.experimental import pallas as pl


def kernel(x, gate_W, gate_b, fc1_W, fc1_b, fc2_W, fc2_b):
    raise NotImplementedError("write your pallas kernel here")


# speedup vs baseline: 1.3153x; 1.3153x over previous
import functools

import jax
import jax.numpy as jnp
from jax import lax
from jax.experimental import pallas as pl
from jax.experimental.pallas import tpu as pltpu
from jax.experimental.pallas import tpu_sc as plsc

N = 2048
D = 768
E = 16
FF = 512
K = 2
COEFF = 0.01
TILE = 128
MAX_TILES = 48
PAD = MAX_TILES * TILE

NW = 32
_SQRT2 = 1.4142135623730951


def _gate_body(x_ref, gw_ref, gb_ref, idx_ref, w_ref, loss_ref):
    x = x_ref[...]
    logits = jnp.dot(x, gw_ref[...], preferred_element_type=jnp.float32)
    logits = logits + gb_ref[...]
    lanes = lax.broadcasted_iota(jnp.int32, (N, E), 1)

    m1 = jnp.max(logits, axis=1, keepdims=True)
    i1 = jnp.min(jnp.where(logits == m1, lanes, E), axis=1, keepdims=True)
    masked = jnp.where(lanes == i1, jnp.float32(-1e30), logits)
    m2 = jnp.max(masked, axis=1, keepdims=True)
    i2 = jnp.min(jnp.where(masked == m2, lanes, E), axis=1, keepdims=True)

    t = jnp.exp(m2 - m1)
    w1 = 1.0 / (1.0 + t)
    idx_ref[...] = jnp.concatenate([i1, i2], axis=1)
    w_ref[...] = jnp.concatenate([w1, 1.0 - w1], axis=1)

    ez = jnp.exp(logits - m1)
    probs = ez / jnp.sum(ez, axis=1, keepdims=True)
    mean_probs = jnp.mean(probs, axis=0, keepdims=True)
    chosen = (lanes == i1).astype(jnp.float32) + (lanes == i2).astype(jnp.float32)
    frac = jnp.mean(chosen, axis=0, keepdims=True)
    loss_ref[...] = COEFF * E * jnp.sum(frac * mean_probs, keepdims=True).reshape(1, 1)


def _gate(x2d, gate_W, gate_b):
    return pl.pallas_call(
        _gate_body,
        out_shape=(
            jax.ShapeDtypeStruct((N, K), jnp.int32),
            jax.ShapeDtypeStruct((N, K), jnp.float32),
            jax.ShapeDtypeStruct((1, 1), jnp.float32),
        ),
    )(x2d, gate_W, gate_b.reshape(1, E))


_G_ROWS = PAD // NW
_G_CHUNK = 96


@functools.lru_cache(maxsize=None)
def _sc_mesh():
    return plsc.VectorSubcoreMesh(core_axis_name="c", subcore_axis_name="s")


@functools.lru_cache(maxsize=None)
def _sc_gather_kernel():
    @functools.partial(
        pl.kernel,
        mesh=_sc_mesh(),
        out_type=jax.ShapeDtypeStruct((PAD, D), jnp.float32),
        scratch_types=[
            pltpu.VMEM((_G_CHUNK,), jnp.int32),
            pltpu.VMEM((_G_CHUNK, D), jnp.float32),
            pltpu.SemaphoreType.DMA,
        ],
    )
    def _sc_gather(x_hbm, map_hbm, out_hbm, idx_v, rows_v, sem):
        wid = lax.axis_index("s") * 2 + lax.axis_index("c")
        base = wid * _G_ROWS
        for c in range(_G_ROWS // _G_CHUNK):
            off = base + c * _G_CHUNK
            pltpu.sync_copy(map_hbm.at[pl.ds(off, _G_CHUNK)], idx_v)
            pltpu.async_copy(x_hbm.at[idx_v], rows_v, sem).wait()
            pltpu.sync_copy(rows_v, out_hbm.at[pl.ds(off, _G_CHUNK)])

    return _sc_gather


def _expert_body(te_ref, xs_ref, w1_ref, b1_ref, w2_ref, b2_ref, wc_ref, ys_ref):
    xg = xs_ref[...]
    h = jnp.dot(xg, w1_ref[0], preferred_element_type=jnp.float32) + b1_ref[0]
    a = h[:, :FF]
    g = h[:, FF:]
    act = a * (0.5 * g * (1.0 + lax.erf(g / _SQRT2)))
    eo = jnp.dot(act, w2_ref[0], preferred_element_type=jnp.float32) + b2_ref[0]
    ys_ref[...] = eo * wc_ref[...]


def _experts(xs, fc1_W, fc1_b, fc2_W, fc2_b, wcol, tile_expert):
    grid_spec = pltpu.PrefetchScalarGridSpec(
        num_scalar_prefetch=1,
        grid=(MAX_TILES,),
        in_specs=[
            pl.BlockSpec((TILE, D), lambda i, s: (i, 0)),
            pl.BlockSpec((1, D, 2 * FF), lambda i, s: (s[i], 0, 0)),
            pl.BlockSpec((1, 1, 2 * FF), lambda i, s: (s[i], 0, 0)),
            pl.BlockSpec((1, FF, D), lambda i, s: (s[i], 0, 0)),
            pl.BlockSpec((1, 1, D), lambda i, s: (s[i], 0, 0)),
            pl.BlockSpec((TILE, 1), lambda i, s: (i, 0)),
        ],
        out_specs=pl.BlockSpec((TILE, D), lambda i, s: (i, 0)),
    )
    return pl.pallas_call(
        _expert_body,
        grid_spec=grid_spec,
        out_shape=jax.ShapeDtypeStruct((PAD, D), jnp.float32),
        compiler_params=pltpu.CompilerParams(
            dimension_semantics=("arbitrary",),
        ),
    )(tile_expert, xs, fc1_W, fc1_b.reshape(E, 1, 2 * FF), fc2_W,
      fc2_b.reshape(E, 1, D), wcol)


_C_TOK = N // NW
_C_CHUNK = 32
_C_VEC = D // 16


@functools.lru_cache(maxsize=None)
def _sc_combine_kernel():
    @functools.partial(
        pl.kernel,
        mesh=_sc_mesh(),
        out_type=jax.ShapeDtypeStruct((N, D), jnp.float32),
        scratch_types=[
            pltpu.VMEM((_C_CHUNK,), jnp.int32),
            pltpu.VMEM((_C_CHUNK,), jnp.int32),
            pltpu.VMEM((_C_CHUNK, D), jnp.float32),
            pltpu.VMEM((_C_CHUNK, D), jnp.float32),
            pltpu.SemaphoreType.DMA,
            pltpu.SemaphoreType.DMA,
        ],
    )
    def _sc_combine(ys_hbm, d0_hbm, d1_hbm, out_hbm, i0_v, i1_v, r0_v, r1_v, s0, s1):
        wid = lax.axis_index("s") * 2 + lax.axis_index("c")
        base = wid * _C_TOK
        for c in range(_C_TOK // _C_CHUNK):
            off = base + c * _C_CHUNK
            pltpu.sync_copy(d0_hbm.at[pl.ds(off, _C_CHUNK)], i0_v)
            pltpu.sync_copy(d1_hbm.at[pl.ds(off, _C_CHUNK)], i1_v)
            cp0 = pltpu.async_copy(ys_hbm.at[i0_v], r0_v, s0)
            cp1 = pltpu.async_copy(ys_hbm.at[i1_v], r1_v, s1)
            cp0.wait()
            cp1.wait()

            def add_body(i, carry):
                r = i // _C_VEC
                col = (i % _C_VEC) * 16
                r0_v[r, pl.ds(col, 16)] = (
                    r0_v[r, pl.ds(col, 16)] + r1_v[r, pl.ds(col, 16)])
                return carry

            lax.fori_loop(0, _C_CHUNK * _C_VEC, add_body, 0)
            pltpu.sync_copy(r0_v, out_hbm.at[pl.ds(off, _C_CHUNK)])

    return _sc_combine


def _routing_metadata(idx, w):
    e_flat = idx.reshape(-1)
    onehot = (e_flat[:, None] == jnp.arange(E, dtype=jnp.int32)[None, :]).astype(jnp.int32)
    cum = jnp.cumsum(onehot, axis=0)
    rank = jnp.take_along_axis(cum, e_flat[:, None], axis=1)[:, 0] - 1
    counts = cum[-1]
    tiles_per = (counts + TILE - 1) // TILE
    tile_bound = jnp.cumsum(tiles_per)
    padded_off = (tile_bound - tiles_per) * TILE
    dest = padded_off[e_flat] + rank
    token_map = jnp.zeros((PAD,), jnp.int32).at[dest].set(
        jnp.arange(N * K, dtype=jnp.int32) // K)
    wcol = jnp.zeros((PAD,), jnp.float32).at[dest].set(w.reshape(-1))
    tile_expert = jnp.clip(
        jnp.searchsorted(tile_bound, jnp.arange(MAX_TILES, dtype=jnp.int32),
                         side="right"),
        0, E - 1).astype(jnp.int32)
    d2 = dest.reshape(N, K)
    return token_map, wcol.reshape(PAD, 1), tile_expert, d2[:, 0], d2[:, 1]


def kernel(x, gate_W, gate_b, fc1_W, fc1_b, fc2_W, fc2_b):
    Bb, Tt, C = x.shape
    x2d = x.reshape(N, D)
    idx, w, loss = _gate(x2d, gate_W, gate_b)
    token_map, wcol, tile_expert, d0, d1 = _routing_metadata(idx, w)
    xs = _sc_gather_kernel()(x2d, token_map)
    ys = _experts(xs, fc1_W, fc1_b, fc2_W, fc2_b, wcol, tile_expert)
    out = _sc_combine_kernel()(ys, d0, d1)
    return out.reshape(Bb, Tt, C), loss.reshape(())


# speedup vs baseline: 3.0122x; 2.2901x over previous
import functools

import jax
import jax.numpy as jnp
from jax import lax
from jax.experimental import pallas as pl
from jax.experimental.pallas import tpu as pltpu
from jax.experimental.pallas import tpu_sc as plsc

N = 2048
D = 768
E = 16
FF = 512
K = 2
COEFF = 0.01
TILE = 128
MAX_TILES = 48
PAD = MAX_TILES * TILE

NW = 32
_SQRT2 = 1.4142135623730951


def _gate_body(x_ref, gw_ref, gb_ref, d0_ref, d1_ref, w0_ref, w1_ref,
               te_ref, nt_ref, loss_ref, xpk_ref):
    x = x_ref[...]
    xb = x.astype(jnp.bfloat16)
    lo32 = lax.bitcast_convert_type(xb[:, :D // 2], jnp.uint16).astype(jnp.uint32)
    hi32 = lax.bitcast_convert_type(xb[:, D // 2:], jnp.uint16).astype(jnp.uint32)
    xpk_ref[...] = lax.bitcast_convert_type(lo32 | (hi32 << 16), jnp.int32)
    logits = jnp.dot(x, gw_ref[...], preferred_element_type=jnp.float32)
    logits = logits + gb_ref[...]
    lanes = lax.broadcasted_iota(jnp.int32, (N, E), 1)

    m1 = jnp.max(logits, axis=1, keepdims=True)
    i1 = jnp.min(jnp.where(logits == m1, lanes, E), axis=1, keepdims=True)
    masked = jnp.where(lanes == i1, jnp.float32(-1e30), logits)
    m2 = jnp.max(masked, axis=1, keepdims=True)
    i2 = jnp.min(jnp.where(masked == m2, lanes, E), axis=1, keepdims=True)

    t = jnp.exp(m2 - m1)
    w0_ref[...] = (1.0 / (1.0 + t)).reshape(N // 128, 128)
    w1_ref[...] = (t / (1.0 + t)).reshape(N // 128, 128)

    ez = jnp.exp(logits - m1)
    probs = ez / jnp.sum(ez, axis=1, keepdims=True)
    mean_probs = jnp.mean(probs, axis=0, keepdims=True)
    onehot_a = (lanes == i1).astype(jnp.int32)
    onehot_b = (lanes == i2).astype(jnp.int32)
    chosen = (onehot_a + onehot_b).astype(jnp.float32)
    frac = jnp.mean(chosen, axis=0, keepdims=True)
    loss_ref[...] = COEFF * E * jnp.sum(frac * mean_probs, keepdims=True).reshape(1, 1)

    oh = onehot_a + onehot_b
    cum = oh
    s = 1
    while s < N:
        shifted = jnp.concatenate(
            [jnp.zeros((s, E), jnp.int32), cum[:-s]], axis=0)
        cum = cum + shifted
        s *= 2
    cum_excl = cum - oh
    counts = cum[N - 1:N]
    tiles_per = (counts + (TILE - 1)) >> 7
    bound = tiles_per
    s = 1
    while s < E:
        bshift = jnp.concatenate(
            [jnp.zeros((1, s), jnp.int32), bound[:, :-s]], axis=1)
        bound = bound + bshift
        s *= 2
    padded_off = (bound - tiles_per) * TILE
    rank0 = jnp.sum(onehot_a * cum_excl, axis=1, keepdims=True)
    rank1 = jnp.sum(onehot_b * cum_excl, axis=1, keepdims=True)
    off0 = jnp.sum(onehot_a * padded_off, axis=1, keepdims=True)
    off1 = jnp.sum(onehot_b * padded_off, axis=1, keepdims=True)
    d0_ref[...] = (off0 + rank0).reshape(N // 128, 128)
    d1_ref[...] = (off1 + rank1).reshape(N // 128, 128)
    tiota = lax.broadcasted_iota(jnp.int32, (MAX_TILES, E), 0)
    te_ref[...] = jnp.minimum(
        jnp.sum((jnp.broadcast_to(bound, (MAX_TILES, E)) <= tiota).astype(
            jnp.int32), axis=1, keepdims=True), E - 1)
    nt_ref[...] = bound[:, E - 1:E]


def _gate(x2d, gate_W, gate_b):
    return pl.pallas_call(
        _gate_body,
        out_shape=(
            jax.ShapeDtypeStruct((N // 128, 128), jnp.int32),
            jax.ShapeDtypeStruct((N // 128, 128), jnp.int32),
            jax.ShapeDtypeStruct((N // 128, 128), jnp.float32),
            jax.ShapeDtypeStruct((N // 128, 128), jnp.float32),
            jax.ShapeDtypeStruct((MAX_TILES, 1), jnp.int32),
            jax.ShapeDtypeStruct((1, 1), jnp.int32),
            jax.ShapeDtypeStruct((1, 1), jnp.float32),
            jax.ShapeDtypeStruct((N, D // 2), jnp.int32),
        ),
    )(x2d, gate_W, gate_b.reshape(1, E))


_S_TOK = N // NW
_S_NCH = 2
_S_CH = _S_TOK // _S_NCH


@functools.lru_cache(maxsize=None)
def _sc_mesh():
    return plsc.VectorSubcoreMesh(core_axis_name="c", subcore_axis_name="s")


@functools.lru_cache(maxsize=None)
def _sc_scatter_kernel():
    @functools.partial(
        pl.kernel,
        mesh=_sc_mesh(),
        out_type=jax.ShapeDtypeStruct((PAD, D // 2), jnp.int32),
        scratch_types=[
            pltpu.VMEM((_S_NCH, _S_CH), jnp.int32),
            pltpu.VMEM((_S_NCH, _S_CH), jnp.int32),
            [pltpu.VMEM((_S_CH, D // 2), jnp.int32)] * _S_NCH,
            [pltpu.SemaphoreType.DMA] * _S_NCH,
            [pltpu.SemaphoreType.DMA] * (2 * _S_NCH),
        ],
    )
    def _sc_scatter(x_hbm, d0_hbm, d1_hbm, xs_hbm,
                    i0_v, i1_v, bufs, lsems, ssems):
        wid = lax.axis_index("s") * 2 + lax.axis_index("c")
        lo = wid * _S_TOK
        for c in range(_S_NCH):
            pltpu.sync_copy(d0_hbm.at[pl.ds(lo + c * _S_CH, _S_CH)], i0_v.at[c])
            pltpu.sync_copy(d1_hbm.at[pl.ds(lo + c * _S_CH, _S_CH)], i1_v.at[c])
        ld = {}
        for c in range(_S_NCH):
            ld[c] = pltpu.async_copy(
                x_hbm.at[pl.ds(lo + c * _S_CH, _S_CH)], bufs[c], lsems[c])
        sd = {}
        for c in range(_S_NCH):
            ld[c].wait()
            sd[2 * c] = pltpu.async_copy(
                bufs[c], xs_hbm.at[i0_v.at[c]], ssems[2 * c])
            sd[2 * c + 1] = pltpu.async_copy(
                bufs[c], xs_hbm.at[i1_v.at[c]], ssems[2 * c + 1])
        for c in range(2 * _S_NCH):
            sd[c].wait()

    return _sc_scatter


def _expert_body(te_ref, nt_ref, xs_ref, w1_ref, b1_ref, w2_ref, b2_ref,
                 ys_ref):
    @pl.when(pl.program_id(0) < nt_ref[0])
    def _():
        xu = lax.bitcast_convert_type(xs_ref[...], jnp.uint32)
        xlo = lax.bitcast_convert_type(xu << 16, jnp.float32)
        xhi = lax.bitcast_convert_type(xu & jnp.uint32(0xFFFF0000), jnp.float32)
        xg = jnp.concatenate([xlo, xhi], axis=1)
        h = jnp.dot(xg, w1_ref[0], preferred_element_type=jnp.float32)
        h = h + b1_ref[0]
        a = h[:, :FF]
        g = h[:, FF:]
        act = a * (0.5 * g * (1.0 + lax.erf(g / _SQRT2)))
        eo = jnp.dot(act, w2_ref[0], preferred_element_type=jnp.float32)
        eo = eo + b2_ref[0]
        eb = eo.astype(jnp.bfloat16)
        lo32 = lax.bitcast_convert_type(eb[:, :D // 2], jnp.uint16).astype(jnp.uint32)
        hi32 = lax.bitcast_convert_type(eb[:, D // 2:], jnp.uint16).astype(jnp.uint32)
        ys_ref[...] = lax.bitcast_convert_type(lo32 | (hi32 << 16), jnp.int32)


def _experts(xs, fc1_W, fc1_b, fc2_W, fc2_b, tile_expert, num_tiles):
    grid_spec = pltpu.PrefetchScalarGridSpec(
        num_scalar_prefetch=2,
        grid=(MAX_TILES,),
        in_specs=[
            pl.BlockSpec((TILE, D // 2), lambda i, s, n: (i, 0)),
            pl.BlockSpec((1, D, 2 * FF), lambda i, s, n: (s[i], 0, 0)),
            pl.BlockSpec((1, 1, 2 * FF), lambda i, s, n: (s[i], 0, 0)),
            pl.BlockSpec((1, FF, D), lambda i, s, n: (s[i], 0, 0)),
            pl.BlockSpec((1, 1, D), lambda i, s, n: (s[i], 0, 0)),
        ],
        out_specs=pl.BlockSpec((TILE, D // 2), lambda i, s, n: (i, 0)),
    )
    return pl.pallas_call(
        _expert_body,
        grid_spec=grid_spec,
        out_shape=jax.ShapeDtypeStruct((PAD, D // 2), jnp.int32),
        compiler_params=pltpu.CompilerParams(
            dimension_semantics=("arbitrary",),
        ),
    )(tile_expert, num_tiles, xs, fc1_W, fc1_b.reshape(E, 1, 2 * FF), fc2_W,
      fc2_b.reshape(E, 1, D))


_C_TOK = N // NW
_C_CHUNK = 16
_C_VEC = D // 16


@functools.lru_cache(maxsize=None)
def _sc_combine_kernel():
    @functools.partial(
        pl.kernel,
        mesh=_sc_mesh(),
        out_type=jax.ShapeDtypeStruct((N, D), jnp.float32),
        scratch_types=[
            pltpu.VMEM((_C_TOK // _C_CHUNK, _C_CHUNK), jnp.int32),
            pltpu.VMEM((_C_TOK // _C_CHUNK, _C_CHUNK), jnp.int32),
            pltpu.VMEM((_C_TOK,), jnp.float32),
            pltpu.VMEM((_C_TOK,), jnp.float32),
            [pltpu.VMEM((_C_CHUNK, D // 2), jnp.int32)] * (_C_TOK // _C_CHUNK),
            [pltpu.VMEM((_C_CHUNK, D // 2), jnp.int32)] * (_C_TOK // _C_CHUNK),
            [pltpu.VMEM((_C_CHUNK, D), jnp.float32)] * (_C_TOK // _C_CHUNK),
            [pltpu.SemaphoreType.DMA] * (2 * (_C_TOK // _C_CHUNK)),
            [pltpu.SemaphoreType.DMA] * (_C_TOK // _C_CHUNK),
        ],
    )
    def _sc_combine(ys_hbm, d0_hbm, d1_hbm, w0_hbm, w1_hbm, out_hbm,
                    i0_v, i1_v, w0_v, w1_v, r0s, r1s, obufs, gsems, wsems):
        wid = lax.axis_index("s") * 2 + lax.axis_index("c")
        base = wid * _C_TOK
        nch = _C_TOK // _C_CHUNK
        for c in range(nch):
            pltpu.sync_copy(
                d0_hbm.at[pl.ds(base + c * _C_CHUNK, _C_CHUNK)], i0_v.at[c])
            pltpu.sync_copy(
                d1_hbm.at[pl.ds(base + c * _C_CHUNK, _C_CHUNK)], i1_v.at[c])
        pltpu.sync_copy(w0_hbm.at[pl.ds(base, _C_TOK)], w0_v)
        pltpu.sync_copy(w1_hbm.at[pl.ds(base, _C_TOK)], w1_v)
        g0 = {}
        g1 = {}
        for c in range(nch):
            g0[c] = pltpu.async_copy(
                ys_hbm.at[i0_v.at[c]], r0s[c], gsems[2 * c])
            g1[c] = pltpu.async_copy(
                ys_hbm.at[i1_v.at[c]], r1s[c], gsems[2 * c + 1])
        dn = lax.GatherDimensionNumbers(
            offset_dims=(), collapsed_slice_dims=(0,), start_index_map=(0,))
        wd = {}
        for c in range(nch):
            g0[c].wait()
            g1[c].wait()
            wv0 = w0_v[pl.ds(c * _C_CHUNK, 16)]
            wv1 = w1_v[pl.ds(c * _C_CHUNK, 16)]

            def rbody(r, carry, c=c, wv0=wv0, wv1=wv1):
                ridx = jnp.full((16, 1), r, jnp.int32)
                sc0 = lax.gather(wv0, ridx, dn, (1,),
                                 mode=lax.GatherScatterMode.PROMISE_IN_BOUNDS)
                sc1 = lax.gather(wv1, ridx, dn, (1,),
                                 mode=lax.GatherScatterMode.PROMISE_IN_BOUNDS)
                hmask = jnp.full((16,), 0xFFFF0000, jnp.uint32)

                def sbody(j, carry2):
                    col = j * 16
                    au = lax.bitcast_convert_type(
                        r0s[c][r, pl.ds(col, 16)], jnp.uint32)
                    bu = lax.bitcast_convert_type(
                        r1s[c][r, pl.ds(col, 16)], jnp.uint32)
                    alo = lax.bitcast_convert_type(au << 16, jnp.float32)
                    ahi = lax.bitcast_convert_type(au & hmask, jnp.float32)
                    blo = lax.bitcast_convert_type(bu << 16, jnp.float32)
                    bhi = lax.bitcast_convert_type(bu & hmask, jnp.float32)
                    obufs[c][r, pl.ds(col, 16)] = sc0 * alo + sc1 * blo
                    obufs[c][r, pl.ds(D // 2 + col, 16)] = sc0 * ahi + sc1 * bhi
                    return carry2

                lax.fori_loop(0, D // 32, sbody, 0)
                return carry

            lax.fori_loop(0, _C_CHUNK, rbody, 0)
            wd[c] = pltpu.async_copy(
                obufs[c], out_hbm.at[pl.ds(base + c * _C_CHUNK, _C_CHUNK)],
                wsems[c])
        for c in range(nch):
            wd[c].wait()

    return _sc_combine


def kernel(x, gate_W, gate_b, fc1_W, fc1_b, fc2_W, fc2_b):
    Bb, Tt, C = x.shape
    x2d = x.reshape(N, D)
    d0, d1, w0, w1, te, nt, loss, xbf = _gate(x2d, gate_W, gate_b)
    d0 = d0.reshape(N)
    d1 = d1.reshape(N)
    xs = _sc_scatter_kernel()(xbf, d0, d1)
    ys = _experts(xs, fc1_W, fc1_b, fc2_W, fc2_b, te.reshape(MAX_TILES),
                  nt.reshape(1))
    out = _sc_combine_kernel()(ys, d0, d1, w0.reshape(N), w1.reshape(N))
    return out.reshape(Bb, Tt, C), loss.reshape(())
